# baseline (speedup 1.0000x reference)
import functools

import jax
import jax.numpy as jnp
from jax import lax
from jax.experimental import pallas as pl
from jax.experimental.pallas import tpu as pltpu
from jax.experimental.pallas import tpu_sc as plsc

_COMMITMENT = 0.25


def _dist_kernel(xn_ref, cn_ref, xm2_ref, cb_ref, idx_ref, *, C, T):
    xm2 = xm2_ref[0]
    cb = cb_ref[0]
    xn = xn_ref[0]
    cn = cn_ref[0]

    cross = jax.lax.dot_general(cb, xm2, (((1,), (0,)), ((), ())),
                                preferred_element_type=jnp.float32)
    dist = (xn + cn) + cross

    m = jnp.min(dist, axis=0, keepdims=True)
    iota = jax.lax.broadcasted_iota(jnp.int32, dist.shape, 0)
    idx = jnp.min(jnp.where(dist == m, iota, C), axis=0, keepdims=True)
    idx_ref[...] = idx[None]


def _sc_body(cb_hbm, xt_hbm, idx_hbm, qst_hbm, counts_hbm, loss_hbm,
             idx_v, gidx_v, hidx_v, ones_v, rows_v, x_v, zer_v, lacc_v,
             shared, sem, *, C, d, T):
    core = lax.axis_index("c")
    sub = lax.axis_index("s")
    kk = sub // 4
    k = core * 4 + kk
    b = sub % 4

    pltpu.sync_copy(idx_hbm.at[k, 0, pl.ds(b * T, T)], idx_v)

    zeros16 = jnp.zeros((16,), jnp.float32)
    zslice = 4 * C // 16

    def zbody(i, _):
        zer_v[pl.ds(i * 16, 16)] = zeros16
        return 0

    lax.fori_loop(0, zslice // 16, zbody, 0)
    pltpu.sync_copy(zer_v, shared.at[pl.ds(sub * zslice, zslice)])

    ones16 = jnp.ones((16,), jnp.float32)
    for i in range(T // 16):
        iv = idx_v[pl.ds(i * 16, 16)]
        gidx_v[pl.ds(i * 16, 16)] = iv + k * C
        hidx_v[pl.ds(i * 16, 16)] = iv + kk * C
        ones_v[pl.ds(i * 16, 16)] = ones16

    pltpu.async_copy(cb_hbm.at[gidx_v], rows_v, sem).wait()

    plsc.subcore_barrier()
    pltpu.sync_copy(ones_v, shared.at[hidx_v], add=True)

    pltpu.sync_copy(xt_hbm.at[b, :, pl.ds(k * d, d)], x_v)

    lacc_v[...] = jnp.zeros((16,), jnp.float32)

    def body(t, _):
        q0 = rows_v[t, pl.ds(0, 16)]
        x0 = x_v[t, pl.ds(0, 16)]
        e0 = q0 - x0
        x_v[t, pl.ds(0, 16)] = x0 + e0
        q1 = rows_v[t, pl.ds(16, 16)]
        x1 = x_v[t, pl.ds(16, 16)]
        e1 = q1 - x1
        x_v[t, pl.ds(16, 16)] = x1 + e1
        lacc_v[...] = lacc_v[...] + e0 * e0 + e1 * e1
        return 0

    lax.fori_loop(0, T, body, 0)
    pltpu.sync_copy(x_v, qst_hbm.at[b, :, pl.ds(k * d, d)])
    pltpu.sync_copy(lacc_v, loss_hbm.at[core * 16 + sub])

    plsc.subcore_barrier()

    @pl.when(b == 0)
    def _():
        pltpu.sync_copy(shared.at[pl.ds(kk * C, C)], counts_hbm.at[k])


def _fin_kernel(counts_ref, loss_ref, perp_ref, commit_ref, *, B, K, d, T):
    cnts = counts_ref[...]
    p = cnts * (1.0 / (B * T))
    ent = -jnp.sum(p * jnp.log(p + 1e-8), axis=1, keepdims=True)
    perp_ref[...] = jnp.exp(ent)
    n = B * K * d * T
    commit_ref[...] = (jnp.sum(loss_ref[...])
                       * ((1.0 + _COMMITMENT) / n)).reshape(1, 1)


def kernel(x, codebooks):
    B, D, T = x.shape
    K, C, d = codebooks.shape
    x_chunks = x.reshape(B, K, d, T)
    x_norm = jnp.sum(x_chunks ** 2, axis=2)[:, :, None, :]
    c_norm = jnp.sum(codebooks ** 2, axis=2)[:, :, None]
    xm2 = -2.0 * x_chunks

    BT = B * T
    xn_t = jnp.transpose(x_norm[:, :, 0, :], (1, 0, 2)).reshape(K, 1, BT)
    xm2_t = jnp.transpose(xm2, (1, 2, 0, 3)).reshape(K, d, BT)

    dk = functools.partial(_dist_kernel, C=C, T=BT)
    idx = pl.pallas_call(
        dk,
        grid=(K,),
        in_specs=[
            pl.BlockSpec((1, 1, BT), lambda k: (k, 0, 0)),
            pl.BlockSpec((1, C, 1), lambda k: (k, 0, 0)),
            pl.BlockSpec((1, d, BT), lambda k: (k, 0, 0)),
            pl.BlockSpec((1, C, d), lambda k: (k, 0, 0)),
        ],
        out_specs=pl.BlockSpec((1, 1, BT), lambda k: (k, 0, 0)),
        out_shape=jax.ShapeDtypeStruct((K, 1, BT), jnp.int32),
        compiler_params=pltpu.CompilerParams(
            dimension_semantics=("parallel",),
        ),
    )(xn_t, c_norm, xm2_t, codebooks)

    xt = jnp.transpose(x, (0, 2, 1))
    cb_flat = codebooks.reshape(K * C, d)

    mesh = plsc.VectorSubcoreMesh(core_axis_name="c", subcore_axis_name="s")
    sc = functools.partial(
        pl.kernel,
        mesh=mesh,
        out_type=[
            jax.ShapeDtypeStruct((B, T, D), jnp.float32),
            jax.ShapeDtypeStruct((K, C), jnp.float32),
            jax.ShapeDtypeStruct((32, 16), jnp.float32),
        ],
        scratch_types=[
            pltpu.VMEM((T,), jnp.int32),
            pltpu.VMEM((T,), jnp.int32),
            pltpu.VMEM((T,), jnp.int32),
            pltpu.VMEM((T,), jnp.float32),
            pltpu.VMEM((T, d), jnp.float32),
            pltpu.VMEM((T, d), jnp.float32),
            pltpu.VMEM((4 * C // 16,), jnp.float32),
            pltpu.VMEM((16,), jnp.float32),
            pltpu.VMEM_SHARED((4 * C,), jnp.float32),
            pltpu.SemaphoreType.DMA,
        ],
        compiler_params=pltpu.CompilerParams(use_tc_tiling_on_sc=False),
    )(functools.partial(_sc_body, C=C, d=d, T=T))
    qst_t, counts, loss_parts = sc(cb_flat, xt, idx)

    fk = functools.partial(_fin_kernel, B=B, K=K, d=d, T=T)
    perp, commit = pl.pallas_call(
        fk,
        in_specs=[
            pl.BlockSpec((K, C), lambda: (0, 0)),
            pl.BlockSpec((32, 16), lambda: (0, 0)),
        ],
        out_specs=[
            pl.BlockSpec((K, 1), lambda: (0, 0)),
            pl.BlockSpec((1, 1), lambda: (0, 0)),
        ],
        out_shape=[
            jax.ShapeDtypeStruct((K, 1), jnp.float32),
            jax.ShapeDtypeStruct((1, 1), jnp.float32),
        ],
    )(counts.reshape(K, C), loss_parts)

    return jnp.transpose(qst_t, (0, 2, 1)), commit[0, 0], perp.reshape(K)


# speedup vs baseline: 1.0110x; 1.0110x over previous
import functools

import jax
import jax.numpy as jnp
from jax import lax
from jax.experimental import pallas as pl
from jax.experimental.pallas import tpu as pltpu
from jax.experimental.pallas import tpu_sc as plsc

_COMMITMENT = 0.25


def _dist_kernel(xn_ref, cn_ref, xm2_ref, cb_ref, idx_ref, *, C, T):
    xm2 = xm2_ref[0]
    cb = cb_ref[0]
    xn = xn_ref[0]
    cn = cn_ref[0]

    cross = jax.lax.dot_general(cb, xm2, (((1,), (0,)), ((), ())),
                                preferred_element_type=jnp.float32)
    dist = (xn + cn) + cross

    m = jnp.min(dist, axis=0, keepdims=True)
    iota_c = jax.lax.broadcasted_iota(
        jnp.int32, (dist.shape[0], 1), 0).astype(jnp.float32)
    sel = jnp.where(dist == m, iota_c, float(C))
    idx = jnp.min(sel, axis=0, keepdims=True).astype(jnp.int32)
    idx_ref[...] = idx[None]


def _sc_body(cb_hbm, xt_hbm, idx_hbm, qst_hbm, counts_hbm, loss_hbm,
             idx_v, gidx_v, hidx_v, ones_v, rows_v, x_v, zer_v, lacc_v,
             shared, sem, *, C, d, T):
    core = lax.axis_index("c")
    sub = lax.axis_index("s")
    kk = sub // 4
    k = core * 4 + kk
    b = sub % 4

    pltpu.sync_copy(idx_hbm.at[k, 0, pl.ds(b * T, T)], idx_v)

    zeros16 = jnp.zeros((16,), jnp.float32)
    zslice = 4 * C // 16

    def zbody(i, _):
        zer_v[pl.ds(i * 16, 16)] = zeros16
        return 0

    lax.fori_loop(0, zslice // 16, zbody, 0)
    pltpu.sync_copy(zer_v, shared.at[pl.ds(sub * zslice, zslice)])

    ones16 = jnp.ones((16,), jnp.float32)
    for i in range(T // 16):
        iv = idx_v[pl.ds(i * 16, 16)]
        gidx_v[pl.ds(i * 16, 16)] = iv + k * C
        hidx_v[pl.ds(i * 16, 16)] = iv + kk * C
        ones_v[pl.ds(i * 16, 16)] = ones16

    pltpu.async_copy(cb_hbm.at[gidx_v], rows_v, sem).wait()

    plsc.subcore_barrier()
    pltpu.sync_copy(ones_v, shared.at[hidx_v], add=True)

    pltpu.sync_copy(xt_hbm.at[b, :, pl.ds(k * d, d)], x_v)

    lacc_v[...] = jnp.zeros((16,), jnp.float32)

    def body(t, _):
        q0 = rows_v[t, pl.ds(0, 16)]
        x0 = x_v[t, pl.ds(0, 16)]
        e0 = q0 - x0
        x_v[t, pl.ds(0, 16)] = x0 + e0
        q1 = rows_v[t, pl.ds(16, 16)]
        x1 = x_v[t, pl.ds(16, 16)]
        e1 = q1 - x1
        x_v[t, pl.ds(16, 16)] = x1 + e1
        lacc_v[...] = lacc_v[...] + e0 * e0 + e1 * e1
        return 0

    lax.fori_loop(0, T, body, 0)
    pltpu.sync_copy(x_v, qst_hbm.at[b, :, pl.ds(k * d, d)])
    pltpu.sync_copy(lacc_v, loss_hbm.at[core * 16 + sub])

    plsc.subcore_barrier()

    @pl.when(b == 0)
    def _():
        pltpu.sync_copy(shared.at[pl.ds(kk * C, C)], counts_hbm.at[k])


def _fin_kernel(counts_ref, loss_ref, perp_ref, commit_ref, *, B, K, d, T):
    cnts = counts_ref[...]
    p = cnts * (1.0 / (B * T))
    ent = -jnp.sum(p * jnp.log(p + 1e-8), axis=1, keepdims=True)
    perp_ref[...] = jnp.exp(ent)
    n = B * K * d * T
    commit_ref[...] = (jnp.sum(loss_ref[...])
                       * ((1.0 + _COMMITMENT) / n)).reshape(1, 1)


def kernel(x, codebooks):
    B, D, T = x.shape
    K, C, d = codebooks.shape
    x_chunks = x.reshape(B, K, d, T)
    x_norm = jnp.sum(x_chunks ** 2, axis=2)[:, :, None, :]
    c_norm = jnp.sum(codebooks ** 2, axis=2)[:, :, None]
    xm2 = -2.0 * x_chunks

    BT = B * T
    xn_t = jnp.transpose(x_norm[:, :, 0, :], (1, 0, 2)).reshape(K, 1, BT)
    xm2_t = jnp.transpose(xm2, (1, 2, 0, 3)).reshape(K, d, BT)

    dk = functools.partial(_dist_kernel, C=C, T=BT)
    idx = pl.pallas_call(
        dk,
        grid=(K,),
        in_specs=[
            pl.BlockSpec((1, 1, BT), lambda k: (k, 0, 0)),
            pl.BlockSpec((1, C, 1), lambda k: (k, 0, 0)),
            pl.BlockSpec((1, d, BT), lambda k: (k, 0, 0)),
            pl.BlockSpec((1, C, d), lambda k: (k, 0, 0)),
        ],
        out_specs=pl.BlockSpec((1, 1, BT), lambda k: (k, 0, 0)),
        out_shape=jax.ShapeDtypeStruct((K, 1, BT), jnp.int32),
    )(xn_t, c_norm, xm2_t, codebooks)

    xt = jnp.transpose(x, (0, 2, 1))
    cb_flat = codebooks.reshape(K * C, d)

    mesh = plsc.VectorSubcoreMesh(core_axis_name="c", subcore_axis_name="s")
    sc = functools.partial(
        pl.kernel,
        mesh=mesh,
        out_type=[
            jax.ShapeDtypeStruct((B, T, D), jnp.float32),
            jax.ShapeDtypeStruct((K, C), jnp.float32),
            jax.ShapeDtypeStruct((32, 16), jnp.float32),
        ],
        scratch_types=[
            pltpu.VMEM((T,), jnp.int32),
            pltpu.VMEM((T,), jnp.int32),
            pltpu.VMEM((T,), jnp.int32),
            pltpu.VMEM((T,), jnp.float32),
            pltpu.VMEM((T, d), jnp.float32),
            pltpu.VMEM((T, d), jnp.float32),
            pltpu.VMEM((4 * C // 16,), jnp.float32),
            pltpu.VMEM((16,), jnp.float32),
            pltpu.VMEM_SHARED((4 * C,), jnp.float32),
            pltpu.SemaphoreType.DMA,
        ],
        compiler_params=pltpu.CompilerParams(use_tc_tiling_on_sc=False),
    )(functools.partial(_sc_body, C=C, d=d, T=T))
    qst_t, counts, loss_parts = sc(cb_flat, xt, idx)

    fk = functools.partial(_fin_kernel, B=B, K=K, d=d, T=T)
    perp, commit = pl.pallas_call(
        fk,
        in_specs=[
            pl.BlockSpec((K, C), lambda: (0, 0)),
            pl.BlockSpec((32, 16), lambda: (0, 0)),
        ],
        out_specs=[
            pl.BlockSpec((K, 1), lambda: (0, 0)),
            pl.BlockSpec((1, 1), lambda: (0, 0)),
        ],
        out_shape=[
            jax.ShapeDtypeStruct((K, 1), jnp.float32),
            jax.ShapeDtypeStruct((1, 1), jnp.float32),
        ],
    )(counts.reshape(K, C), loss_parts)

    return jnp.transpose(qst_t, (0, 2, 1)), commit[0, 0], perp.reshape(K)
